# baseline (speedup 1.0000x reference)
import functools

import jax
import jax.numpy as jnp
from jax import lax
from jax.experimental import pallas as pl
from jax.experimental.pallas import tpu as pltpu
from jax.experimental.pallas import tpu_sc as plsc

N = 10000
F = 128
E = 320000
EPAD = 327680
NACC = 10240
DUMMY = N + 16
STRIPE = NACC // 16
IC = 40
NCI = STRIPE // IC
DEG_CH = 128
DEG_NCH = EPAD // DEG_CH
DEG_STRIPE = NACC // 16
CH = 64
NCH = EPAD // CH
NCHT = NCH // 16
NBUF = 4
G = 40
NGRP = NCHT // G

_mesh = plsc.VectorSubcoreMesh(core_axis_name="c", subcore_axis_name="s",
                               num_cores=2, num_subcores=16)


def _deg_body(dst_hbm, out_hbm, ones_v, idx_v, zero_v, acc_sh):
    c = lax.axis_index("c")
    s = lax.axis_index("s")

    @pl.loop(0, DEG_STRIPE, step=16)
    def _(i):
        zero_v[pl.ds(i, 16)] = jnp.zeros((16,), jnp.float32)

    @pl.loop(0, DEG_CH, step=16)
    def _(i):
        ones_v[pl.ds(i, 16)] = jnp.ones((16,), jnp.float32)

    pltpu.sync_copy(zero_v, acc_sh.at[pl.ds(s * DEG_STRIPE, DEG_STRIPE)])
    plsc.subcore_barrier()

    base = (c * 16 + s) * (DEG_NCH // 32)

    @pl.loop(0, DEG_NCH // 32)
    def _(j):
        pltpu.sync_copy(dst_hbm.at[base + j], idx_v)
        pltpu.sync_copy(ones_v, acc_sh.at[idx_v.at[0]], add=True)

    plsc.subcore_barrier()
    pltpu.sync_copy(acc_sh.at[pl.ds(s * DEG_STRIPE, DEG_STRIPE)],
                    out_hbm.at[pl.ds(c * NACC + s * DEG_STRIPE, DEG_STRIPE)])


def _deg_counts(dst_r):
    k = pl.kernel(
        _deg_body,
        out_type=jax.ShapeDtypeStruct((2 * NACC,), jnp.float32),
        mesh=_mesh,
        scratch_types=[
            pltpu.VMEM((DEG_CH,), jnp.float32),
            pltpu.VMEM((1, DEG_CH), jnp.int32),
            pltpu.VMEM((DEG_STRIPE,), jnp.float32),
            pltpu.VMEM_SHARED((NACC,), jnp.float32),
        ],
    )
    return k(dst_r).reshape(2, NACC)


def _s_body(u_hbm, srcidx_hbm, dstidx_hbm, v_hbm, sidx_v, didx_v, rows_v,
            gsem, ssem, acc_sh):
    c = lax.axis_index("c")
    s = lax.axis_index("s")
    base = s * NCHT

    for b in range(NBUF):
        r0 = s * STRIPE + b * IC

        @pl.when(r0 < N)
        def _():
            pltpu.async_copy(u_hbm.at[pl.ds(c * N + r0, IC)],
                             rows_v.at[b, pl.ds(0, IC)], gsem.at[b])

    @pl.loop(0, NCI, step=NBUF)
    def _(k):
        for b in range(NBUF):
            ci = k + b
            r0 = s * STRIPE + ci * IC

            @pl.when(r0 < N)
            def _():
                pltpu.make_async_copy(u_hbm.at[pl.ds(c * N + r0, IC)],
                                      rows_v.at[b, pl.ds(0, IC)],
                                      gsem.at[b]).wait()
                pltpu.sync_copy(rows_v.at[b, pl.ds(0, IC)],
                                acc_sh.at[pl.ds(r0, IC)])
                nr = r0 + NBUF * IC

                @pl.when(jnp.logical_and(ci + NBUF < NCI, nr < N))
                def _():
                    pltpu.async_copy(u_hbm.at[pl.ds(c * N + nr, IC)],
                                     rows_v.at[b, pl.ds(0, IC)], gsem.at[b])

    plsc.subcore_barrier()

    for g in range(NGRP):
        pltpu.sync_copy(srcidx_hbm.at[c, pl.ds(base + g * G, G)], sidx_v)
        pltpu.sync_copy(dstidx_hbm.at[pl.ds(base + g * G, G)], didx_v)

        for b in range(NBUF - 1):
            pltpu.async_copy(u_hbm.at[sidx_v.at[b, 0]], rows_v.at[b],
                             gsem.at[b])

        @pl.loop(0, G, step=NBUF)
        def _(j0):
            for b in range(NBUF):
                j = j0 + b
                f = j + NBUF - 1
                bf = (b + NBUF - 1) % NBUF
                pltpu.make_async_copy(u_hbm.at[sidx_v.at[j, 0]], rows_v.at[b],
                                      gsem.at[b]).wait()
                pltpu.async_copy(rows_v.at[b], acc_sh.at[didx_v.at[j, 0]],
                                 ssem.at[b], add=True)

                @pl.when(f < G)
                def _():
                    @pl.when(j >= 1)
                    def _():
                        pltpu.make_async_copy(rows_v.at[bf],
                                              acc_sh.at[didx_v.at[0, 0]],
                                              ssem.at[bf]).wait()

                    pltpu.async_copy(u_hbm.at[sidx_v.at[f, 0]], rows_v.at[bf],
                                     gsem.at[bf])

        for b in range(NBUF):
            pltpu.make_async_copy(rows_v.at[b], acc_sh.at[didx_v.at[0, 0]],
                                  ssem.at[b]).wait()

    plsc.subcore_barrier()

    @pl.loop(0, NCI + NBUF, step=NBUF)
    def _(k):
        for b in range(NBUF):
            ci = k + b
            r0 = s * STRIPE + ci * IC
            rp = r0 - NBUF * IC

            @pl.when(jnp.logical_and(ci >= NBUF, rp < N))
            def _():
                pltpu.make_async_copy(rows_v.at[b, pl.ds(0, IC)],
                                      v_hbm.at[c, pl.ds(rp, IC)],
                                      gsem.at[b]).wait()

            @pl.when(jnp.logical_and(ci < NCI, r0 < N))
            def _():
                pltpu.sync_copy(acc_sh.at[pl.ds(r0, IC)],
                                rows_v.at[b, pl.ds(0, IC)])
                pltpu.async_copy(rows_v.at[b, pl.ds(0, IC)],
                                 v_hbm.at[c, pl.ds(r0, IC)], gsem.at[b])


def _s_apply(u2n, src2, dst_r):
    k = pl.kernel(
        _s_body,
        out_type=jax.ShapeDtypeStruct((2, N, F), jnp.float32),
        mesh=_mesh,
        scratch_types=[
            pltpu.VMEM((G, 1, CH), jnp.int32),
            pltpu.VMEM((G, 1, CH), jnp.int32),
            pltpu.VMEM((NBUF, CH, F), jnp.float32),
            pltpu.SemaphoreType.DMA((NBUF,)),
            pltpu.SemaphoreType.DMA((NBUF,)),
            pltpu.VMEM_SHARED((NACC, F), jnp.float32),
        ],
    )
    return k(u2n, src2, dst_r)


BN = 1000


def _t0_body(x_ref, p_ref, u_ref, dinv_ref):
    p = p_ref[...]
    deg = 1.0 + p[0] + p[1]
    dinv = 1.0 / jnp.sqrt(deg)
    u_ref[0] = x_ref[0] * dinv
    dinv_ref[0] = dinv


def _t0(x, pm):
    return pl.pallas_call(
        _t0_body,
        grid=(2, N // BN),
        in_specs=[
            pl.BlockSpec((1, BN, F), lambda c, i: (c, i, 0)),
            pl.BlockSpec((2, BN, 1), lambda c, i: (0, i, 0)),
        ],
        out_specs=[
            pl.BlockSpec((1, BN, F), lambda c, i: (c, i, 0)),
            pl.BlockSpec((1, BN, 1), lambda c, i: (0, i, 0)),
        ],
        out_shape=[
            jax.ShapeDtypeStruct((2, N, F), jnp.float32),
            jax.ShapeDtypeStruct((1, N, 1), jnp.float32),
        ],
    )(x, pm)


def _t12_body(v_ref, d_ref, w_ref, b_ref, u_ref):
    dinv = d_ref[0]
    a = v_ref[0] * dinv
    h = jnp.dot(a, w_ref[...], preferred_element_type=jnp.float32) + b_ref[...]
    u_ref[0] = jnp.maximum(h, 0.0) * dinv


def _t12(v, dinv, w, b):
    return pl.pallas_call(
        _t12_body,
        grid=(2, N // BN),
        in_specs=[
            pl.BlockSpec((1, BN, F), lambda c, i: (c, i, 0)),
            pl.BlockSpec((1, BN, 1), lambda c, i: (0, i, 0)),
            pl.BlockSpec((F, F), lambda c, i: (0, 0)),
            pl.BlockSpec((1, F), lambda c, i: (0, 0)),
        ],
        out_specs=pl.BlockSpec((1, BN, F), lambda c, i: (c, i, 0)),
        out_shape=jax.ShapeDtypeStruct((2, N, F), jnp.float32),
    )(v, dinv, w, b)


def _t3_body(v_ref, d_ref, w1_ref, b1_ref, w2_ref, b2_ref, o1_ref, o2_ref):
    a = v_ref[0] * d_ref[0]
    o1_ref[0] = jnp.dot(a, w1_ref[...], preferred_element_type=jnp.float32) + b1_ref[...]
    o2_ref[0] = jnp.dot(a, w2_ref[...], preferred_element_type=jnp.float32) + b2_ref[...]


def _t3(v, dinv, w31, b31, w32, b32):
    return pl.pallas_call(
        _t3_body,
        grid=(2, N // BN),
        in_specs=[
            pl.BlockSpec((1, BN, F), lambda c, i: (c, i, 0)),
            pl.BlockSpec((1, BN, 1), lambda c, i: (0, i, 0)),
            pl.BlockSpec((F, F), lambda c, i: (0, 0)),
            pl.BlockSpec((1, F), lambda c, i: (0, 0)),
            pl.BlockSpec((F, F), lambda c, i: (0, 0)),
            pl.BlockSpec((1, F), lambda c, i: (0, 0)),
        ],
        out_specs=[
            pl.BlockSpec((1, BN, F), lambda c, i: (c, i, 0)),
            pl.BlockSpec((1, BN, F), lambda c, i: (c, i, 0)),
        ],
        out_shape=[
            jax.ShapeDtypeStruct((2, N, F), jnp.float32),
            jax.ShapeDtypeStruct((2, N, F), jnp.float32),
        ],
    )(v, dinv, w31, b31, w32, b32)


def kernel(sp_feat, aug_feat, edge_index, W1, b1, W2, b2, W31, b31, W32, b32):
    src = edge_index[0].astype(jnp.int32)
    dst = edge_index[1].astype(jnp.int32)
    npad = EPAD - E
    src_p = jnp.concatenate([src, jnp.zeros((npad,), jnp.int32)])
    dst_p = jnp.concatenate([dst, jnp.full((npad,), DUMMY, jnp.int32)])
    src2 = jnp.stack([src_p, src_p + N]).reshape(2, NCH, 1, CH)
    dst_r = dst_p.reshape(NCH, 1, CH)
    dst_deg = dst_p.reshape(DEG_NCH, 1, DEG_CH)

    x = jnp.stack([sp_feat, aug_feat])
    b1r, b2r = b1.reshape(1, F), b2.reshape(1, F)
    b31r, b32r = b31.reshape(1, F), b32.reshape(1, F)

    p = _deg_counts(dst_deg)
    pm = p[:, :N].reshape(2, N, 1)
    u0, dinv = _t0(x, pm)
    v0 = _s_apply(u0.reshape(2 * N, F), src2, dst_r)
    u1 = _t12(v0, dinv, W1, b1r)
    v1 = _s_apply(u1.reshape(2 * N, F), src2, dst_r)
    u2 = _t12(v1, dinv, W2, b2r)
    v2 = _s_apply(u2.reshape(2 * N, F), src2, dst_r)
    o1, o2 = _t3(v2, dinv, W31, b31r, W32, b32r)
    return o1[0], o2[0], o1[1], o2[1]


# speedup vs baseline: 2.4094x; 2.4094x over previous
import functools

import jax
import jax.numpy as jnp
from jax import lax
from jax.experimental import pallas as pl
from jax.experimental.pallas import tpu as pltpu
from jax.experimental.pallas import tpu_sc as plsc

N = 10000
F = 128
E = 320000
EPAD = 327680
NACC = 10240
DUMMY = N + 16
STRIPE = NACC // 16
IC = 40
NCI = STRIPE // IC
DEG_CH = 128
DEG_NCH = EPAD // DEG_CH
DEG_STRIPE = NACC // 16
CH = 64
NCH = EPAD // CH
NCHT = NCH // 16
NBUF = 4
G = 40
NGRP = NCHT // G

_mesh = plsc.VectorSubcoreMesh(core_axis_name="c", subcore_axis_name="s",
                               num_cores=2, num_subcores=16)


def _deg_body(dst_hbm, out_hbm, ones_v, idx_v, zero_v, acc_sh):
    c = lax.axis_index("c")
    s = lax.axis_index("s")

    @pl.loop(0, DEG_STRIPE, step=16)
    def _(i):
        zero_v[pl.ds(i, 16)] = jnp.zeros((16,), jnp.float32)

    @pl.loop(0, DEG_CH, step=16)
    def _(i):
        ones_v[pl.ds(i, 16)] = jnp.ones((16,), jnp.float32)

    pltpu.sync_copy(zero_v, acc_sh.at[pl.ds(s * DEG_STRIPE, DEG_STRIPE)])
    plsc.subcore_barrier()

    base = (c * 16 + s) * (DEG_NCH // 32)

    @pl.loop(0, DEG_NCH // 32)
    def _(j):
        pltpu.sync_copy(dst_hbm.at[base + j], idx_v)
        pltpu.sync_copy(ones_v, acc_sh.at[idx_v.at[0]], add=True)

    plsc.subcore_barrier()
    pltpu.sync_copy(acc_sh.at[pl.ds(s * DEG_STRIPE, DEG_STRIPE)],
                    out_hbm.at[pl.ds(c * NACC + s * DEG_STRIPE, DEG_STRIPE)])


def _deg_counts(dst_r):
    k = pl.kernel(
        _deg_body,
        out_type=jax.ShapeDtypeStruct((2 * NACC,), jnp.float32),
        mesh=_mesh,
        scratch_types=[
            pltpu.VMEM((DEG_CH,), jnp.float32),
            pltpu.VMEM((1, DEG_CH), jnp.int32),
            pltpu.VMEM((DEG_STRIPE,), jnp.float32),
            pltpu.VMEM_SHARED((NACC,), jnp.float32),
        ],
    )
    return k(dst_r).reshape(2, NACC)


def _s_body(u_hbm, srcidx_hbm, dstidx_hbm, v_hbm, sidx_v, didx_v, rows_v,
            gsem, ssem, acc_sh):
    c = lax.axis_index("c")
    s = lax.axis_index("s")
    base = s * NCHT

    for b in range(NBUF):
        r0 = s * STRIPE + b * IC

        @pl.when(r0 < N)
        def _():
            pltpu.async_copy(u_hbm.at[pl.ds(c * N + r0, IC)],
                             rows_v.at[b, pl.ds(0, IC)], gsem.at[b])

    @pl.loop(0, NCI, step=NBUF)
    def _(k):
        for b in range(NBUF):
            ci = k + b
            r0 = s * STRIPE + ci * IC

            @pl.when(r0 < N)
            def _():
                pltpu.make_async_copy(u_hbm.at[pl.ds(c * N + r0, IC)],
                                      rows_v.at[b, pl.ds(0, IC)],
                                      gsem.at[b]).wait()
                pltpu.sync_copy(rows_v.at[b, pl.ds(0, IC)],
                                acc_sh.at[pl.ds(r0, IC)])
                nr = r0 + NBUF * IC

                @pl.when(jnp.logical_and(ci + NBUF < NCI, nr < N))
                def _():
                    pltpu.async_copy(u_hbm.at[pl.ds(c * N + nr, IC)],
                                     rows_v.at[b, pl.ds(0, IC)], gsem.at[b])

    plsc.subcore_barrier()

    for g in range(NGRP):
        pltpu.sync_copy(srcidx_hbm.at[c, pl.ds(base + g * G, G)], sidx_v)
        pltpu.sync_copy(dstidx_hbm.at[pl.ds(base + g * G, G)], didx_v)

        for b in range(NBUF - 1):
            pltpu.async_copy(u_hbm.at[sidx_v.at[b, 0]], rows_v.at[b],
                             gsem.at[b])

        @pl.loop(0, G, step=NBUF)
        def _(j0):
            for b in range(NBUF):
                j = j0 + b
                f = j + NBUF - 1
                bf = (b + NBUF - 1) % NBUF
                pltpu.make_async_copy(u_hbm.at[sidx_v.at[j, 0]], rows_v.at[b],
                                      gsem.at[b]).wait()
                pltpu.async_copy(rows_v.at[b], acc_sh.at[didx_v.at[j, 0]],
                                 ssem.at[b], add=True)

                @pl.when(f < G)
                def _():
                    @pl.when(j >= 1)
                    def _():
                        pltpu.make_async_copy(rows_v.at[bf],
                                              acc_sh.at[didx_v.at[0, 0]],
                                              ssem.at[bf]).wait()

                    pltpu.async_copy(u_hbm.at[sidx_v.at[f, 0]], rows_v.at[bf],
                                     gsem.at[bf])

        for b in range(NBUF):
            pltpu.make_async_copy(rows_v.at[b], acc_sh.at[didx_v.at[0, 0]],
                                  ssem.at[b]).wait()

    plsc.subcore_barrier()

    @pl.loop(0, NCI + NBUF, step=NBUF)
    def _(k):
        for b in range(NBUF):
            ci = k + b
            r0 = s * STRIPE + ci * IC
            rp = r0 - NBUF * IC

            @pl.when(jnp.logical_and(ci >= NBUF, rp < N))
            def _():
                pltpu.make_async_copy(rows_v.at[b, pl.ds(0, IC)],
                                      v_hbm.at[c, pl.ds(rp, IC)],
                                      gsem.at[b]).wait()

            @pl.when(jnp.logical_and(ci < NCI, r0 < N))
            def _():
                pltpu.sync_copy(acc_sh.at[pl.ds(r0, IC)],
                                rows_v.at[b, pl.ds(0, IC)])
                pltpu.async_copy(rows_v.at[b, pl.ds(0, IC)],
                                 v_hbm.at[c, pl.ds(r0, IC)], gsem.at[b])


def _s_apply(u2n, src2, dst_r):
    k = pl.kernel(
        _s_body,
        out_type=jax.ShapeDtypeStruct((2, N, F), jnp.float32),
        mesh=_mesh,
        scratch_types=[
            pltpu.VMEM((G, 1, CH), jnp.int32),
            pltpu.VMEM((G, 1, CH), jnp.int32),
            pltpu.VMEM((NBUF, CH, F), jnp.float32),
            pltpu.SemaphoreType.DMA((NBUF,)),
            pltpu.SemaphoreType.DMA((NBUF,)),
            pltpu.VMEM_SHARED((NACC, F), jnp.float32),
        ],
    )
    return k(u2n, src2, dst_r)


BN = 1000


def _t0_body(x_ref, p_ref, u_ref, dinv_ref):
    p = p_ref[...]
    deg = 1.0 + p[0] + p[1]
    dinv = 1.0 / jnp.sqrt(deg)
    u_ref[0] = x_ref[0] * dinv
    dinv_ref[0] = dinv


def _t0(x, pm):
    return pl.pallas_call(
        _t0_body,
        grid=(2, N // BN),
        in_specs=[
            pl.BlockSpec((1, BN, F), lambda c, i: (c, i, 0)),
            pl.BlockSpec((2, BN, 1), lambda c, i: (0, i, 0)),
        ],
        out_specs=[
            pl.BlockSpec((1, BN, F), lambda c, i: (c, i, 0)),
            pl.BlockSpec((1, BN, 1), lambda c, i: (0, i, 0)),
        ],
        out_shape=[
            jax.ShapeDtypeStruct((2, N, F), jnp.float32),
            jax.ShapeDtypeStruct((1, N, 1), jnp.float32),
        ],
    )(x, pm)


def _t12_body(v_ref, d_ref, w_ref, b_ref, u_ref):
    dinv = d_ref[0]
    a = v_ref[0] * dinv
    h = jnp.dot(a, w_ref[...], preferred_element_type=jnp.float32) + b_ref[...]
    u_ref[0] = jnp.maximum(h, 0.0) * dinv


def _t12(v, dinv, w, b):
    return pl.pallas_call(
        _t12_body,
        grid=(2, N // BN),
        in_specs=[
            pl.BlockSpec((1, BN, F), lambda c, i: (c, i, 0)),
            pl.BlockSpec((1, BN, 1), lambda c, i: (0, i, 0)),
            pl.BlockSpec((F, F), lambda c, i: (0, 0)),
            pl.BlockSpec((1, F), lambda c, i: (0, 0)),
        ],
        out_specs=pl.BlockSpec((1, BN, F), lambda c, i: (c, i, 0)),
        out_shape=jax.ShapeDtypeStruct((2, N, F), jnp.float32),
    )(v, dinv, w, b)


def _t3_body(v_ref, d_ref, w1_ref, b1_ref, w2_ref, b2_ref, o1_ref, o2_ref):
    a = v_ref[0] * d_ref[0]
    o1_ref[0] = jnp.dot(a, w1_ref[...], preferred_element_type=jnp.float32) + b1_ref[...]
    o2_ref[0] = jnp.dot(a, w2_ref[...], preferred_element_type=jnp.float32) + b2_ref[...]


def _t3(v, dinv, w31, b31, w32, b32):
    return pl.pallas_call(
        _t3_body,
        grid=(2, N // BN),
        in_specs=[
            pl.BlockSpec((1, BN, F), lambda c, i: (c, i, 0)),
            pl.BlockSpec((1, BN, 1), lambda c, i: (0, i, 0)),
            pl.BlockSpec((F, F), lambda c, i: (0, 0)),
            pl.BlockSpec((1, F), lambda c, i: (0, 0)),
            pl.BlockSpec((F, F), lambda c, i: (0, 0)),
            pl.BlockSpec((1, F), lambda c, i: (0, 0)),
        ],
        out_specs=[
            pl.BlockSpec((1, BN, F), lambda c, i: (c, i, 0)),
            pl.BlockSpec((1, BN, F), lambda c, i: (c, i, 0)),
        ],
        out_shape=[
            jax.ShapeDtypeStruct((2, N, F), jnp.float32),
            jax.ShapeDtypeStruct((2, N, F), jnp.float32),
        ],
    )(v, dinv, w31, b31, w32, b32)


def kernel(sp_feat, aug_feat, edge_index, W1, b1, W2, b2, W31, b31, W32, b32):
    src = edge_index[0].astype(jnp.int32)
    dst = edge_index[1].astype(jnp.int32)
    npad = EPAD - E
    pad_ids = jnp.arange(npad, dtype=jnp.int32)
    src_p = jnp.concatenate([src, (pad_ids * 37) % N])
    dst_p = jnp.concatenate([dst, N + (pad_ids % (NACC - N - 8))])
    src2 = jnp.stack([src_p, src_p + N]).reshape(2, NCH, 1, CH)
    dst_r = dst_p.reshape(NCH, 1, CH)
    dst_deg = dst_p.reshape(DEG_NCH, 1, DEG_CH)

    x = jnp.stack([sp_feat, aug_feat])
    b1r, b2r = b1.reshape(1, F), b2.reshape(1, F)
    b31r, b32r = b31.reshape(1, F), b32.reshape(1, F)

    p = _deg_counts(dst_deg)
    pm = p[:, :N].reshape(2, N, 1)
    u0, dinv = _t0(x, pm)
    v0 = _s_apply(u0.reshape(2 * N, F), src2, dst_r)
    u1 = _t12(v0, dinv, W1, b1r)
    v1 = _s_apply(u1.reshape(2 * N, F), src2, dst_r)
    u2 = _t12(v1, dinv, W2, b2r)
    v2 = _s_apply(u2.reshape(2 * N, F), src2, dst_r)
    o1, o2 = _t3(v2, dinv, W31, b31r, W32, b32r)
    return o1[0], o2[0], o1[1], o2[1]
